# baseline (speedup 1.0000x reference)
import functools

import jax
import jax.numpy as jnp
import numpy as np
from jax import lax
from jax.experimental import pallas as pl
from jax.experimental.pallas import tpu as pltpu
from jax.experimental.pallas import tpu_sc as plsc

_PRIMES = (499801, 499819, 499853, 499879, 499883, 499897, 499903, 499927,
           499943, 499957, 499969, 499973, 499979, 500009, 500029, 500041)
_NUM_TABLES = 16
_EMBED_DIM = 16
_HIDDEN = 1024
_ORDERS = tuple([2] * 8 + [3] * 8)
_OFFSETS = tuple(np.concatenate([[0], np.cumsum(_PRIMES)[:-1]]).astype(np.int64))

_NC, _NS = 2, 16
_NW = _NC * _NS


def _mod_prime(x, p):
    q = (x.astype(jnp.float32) * np.float32(1.0 / p)).astype(jnp.int32)
    r = x - q * np.int32(p)
    r = jnp.where(r < 0, r + np.int32(p), r)
    r = jnp.where(r >= np.int32(p), r - np.int32(p), r)
    return r


def _hash_kernel(tok0, tok1, tok2, mults, bias, out_ref):
    toks = (tok0, tok1, tok2)
    mask16 = np.int32(0xFFFF)
    _S16 = np.int32(16)
    for t in range(_NUM_TABLES):
        order = _ORDERS[t]
        l0 = jnp.zeros_like(tok0[...])
        l1 = jnp.zeros_like(l0)
        l2 = jnp.zeros_like(l0)
        for p in range(order):
            m = mults[t, p]
            m_lo = m & mask16
            m_hi = lax.shift_right_logical(m, _S16)
            tv = toks[p][...]
            p_lo = m_lo * tv
            p_hi = m_hi * tv
            q0 = p_lo & mask16
            mid = lax.shift_right_logical(p_lo, _S16) + (p_hi & mask16)
            q1 = mid & mask16
            q2 = lax.shift_right_logical(p_hi, _S16) + lax.shift_right_logical(mid, _S16)
            l0 = l0 ^ q0
            l1 = l1 ^ q1
            l2 = l2 ^ q2
        b = bias[t]
        l0 = l0 ^ (b & mask16)
        l1 = l1 ^ lax.shift_right_logical(b, _S16)
        p = _PRIMES[t]
        r1 = _mod_prime(l2 * np.int32(65536) + l1, p)
        r2 = _mod_prime(r1 * np.int32(4096), p)
        idx = _mod_prime(r2 * np.int32(16) + l0, p) + np.int32(_OFFSETS[t])
        nrow, ncol = idx.shape
        for r in range(nrow):
            out_ref[pl.ds((t * nrow + r) * ncol, ncol)] = idx[r]


def _compute_indices(tok0, tok1, tok2, mults, bias):
    P, C = tok0.shape
    return pl.pallas_call(
        _hash_kernel,
        out_shape=jax.ShapeDtypeStruct((_NUM_TABLES * P * C,), jnp.int32),
        in_specs=[
            pl.BlockSpec(memory_space=pltpu.VMEM),
            pl.BlockSpec(memory_space=pltpu.VMEM),
            pl.BlockSpec(memory_space=pltpu.VMEM),
            pl.BlockSpec(memory_space=pltpu.SMEM),
            pl.BlockSpec(memory_space=pltpu.SMEM),
        ],
        out_specs=pl.BlockSpec(memory_space=pltpu.VMEM),
        name="ngram_hash",
    )(tok0, tok1, tok2, mults, bias)


def _gather_body(bs_total, chunk, table_hbm, idx_hbm, out_hbm,
                 idx_v, rows_v, sem):
    rpw = bs_total // 2
    wid = lax.axis_index("s") * np.int32(_NC) + lax.axis_index("c")
    t = wid // np.int32(2)
    h = wid % np.int32(2)
    base = t * np.int32(bs_total) + h * np.int32(rpw)
    pltpu.sync_copy(idx_hbm.at[pl.ds(base, rpw)], idx_v)
    for j in range(rpw // chunk):
        off = np.int32(j * chunk)
        pltpu.make_async_copy(
            table_hbm.at[idx_v.at[pl.ds(off, chunk)]],
            rows_v.at[pl.ds(off, chunk)],
            sem,
        ).start()
    pltpu.make_async_copy(
        table_hbm.at[pl.ds(0, rpw)], rows_v, sem).wait()
    pos0 = h * np.int32(rpw)
    pltpu.sync_copy(
        rows_v,
        out_hbm.at[pl.ds(pos0, rpw),
                   pl.ds(t * np.int32(_EMBED_DIM), _EMBED_DIM)])


def _gather_rows(table, idx_flat):
    n = idx_flat.shape[0]
    bs_total = n // _NUM_TABLES
    chunk = 128
    mesh = plsc.VectorSubcoreMesh(core_axis_name="c", subcore_axis_name="s")
    k = pl.kernel(
        functools.partial(_gather_body, bs_total, chunk),
        out_type=jax.ShapeDtypeStruct(
            (bs_total, _NUM_TABLES * _EMBED_DIM), jnp.float32),
        mesh=mesh,
        name="sc_gather",
        compiler_params=pltpu.CompilerParams(use_tc_tiling_on_sc=False),
        scratch_types=[
            pltpu.VMEM((bs_total // 2,), jnp.int32),
            pltpu.VMEM((bs_total // 2, _EMBED_DIM), jnp.float32),
            pltpu.SemaphoreType.DMA,
        ],
    )
    return k(table, idx_flat)


def _matmul_kernel(emb_ref, w_ref, out_ref):
    out_ref[...] = lax.dot_general(
        emb_ref[...], w_ref[...],
        (((1,), (1,)), ((), ())),
        preferred_element_type=jnp.float32)


def _project(emb, w_out):
    n = emb.shape[0]
    blk = 1024
    return pl.pallas_call(
        _matmul_kernel,
        grid=(n // blk,),
        in_specs=[
            pl.BlockSpec((blk, _NUM_TABLES * _EMBED_DIM),
                         lambda i: (i, np.int32(0))),
            pl.BlockSpec((_HIDDEN, _NUM_TABLES * _EMBED_DIM),
                         lambda i: (np.int32(0), np.int32(0))),
        ],
        out_specs=pl.BlockSpec((blk, _HIDDEN), lambda i: (i, np.int32(0))),
        out_shape=jax.ShapeDtypeStruct((n, _HIDDEN), jnp.float32),
        name="out_proj",
    )(emb, w_out)


def kernel(token_ids, hash_mults, hash_bias, table, w_out):
    B, S = token_ids.shape
    P = B * S // 128
    tok0 = token_ids.astype(jnp.int32)
    tok1 = jnp.pad(tok0[:, :S - 1], ((0, 0), (1, 0)))
    tok2 = jnp.pad(tok0[:, :S - 2], ((0, 0), (2, 0)))
    tok0, tok1, tok2 = (t.reshape(P, 128) for t in (tok0, tok1, tok2))
    mults = hash_mults.astype(jnp.int32)
    bias = hash_bias.astype(jnp.int32)

    idx_flat = _compute_indices(tok0, tok1, tok2, mults, bias)
    emb = _gather_rows(table, idx_flat)
    out = _project(emb, w_out)
    return out.reshape(B, S, _HIDDEN)


# speedup vs baseline: 5.4034x; 5.4034x over previous
import functools

import jax
import jax.numpy as jnp
import numpy as np
from jax import lax
from jax.experimental import pallas as pl
from jax.experimental.pallas import tpu as pltpu
from jax.experimental.pallas import tpu_sc as plsc

_PRIMES = (499801, 499819, 499853, 499879, 499883, 499897, 499903, 499927,
           499943, 499957, 499969, 499973, 499979, 500009, 500029, 500041)
_NUM_TABLES = 16
_EMBED_DIM = 16
_HIDDEN = 1024
_ORDERS = tuple([2] * 8 + [3] * 8)
_OFFSETS = tuple(np.concatenate([[0], np.cumsum(_PRIMES)[:-1]]).astype(np.int64))

_NC, _NS = 2, 16
_NW = _NC * _NS


def _mod_prime(x, p):
    q = (x.astype(jnp.float32) * np.float32(1.0 / p)).astype(jnp.int32)
    r = x - q * np.int32(p)
    r = jnp.where(r < 0, r + np.int32(p), r)
    r = jnp.where(r >= np.int32(p), r - np.int32(p), r)
    return r


def _hash_kernel(tok0, tok1, tok2, mults, bias, out_ref):
    toks = (tok0, tok1, tok2)
    mask16 = np.int32(0xFFFF)
    _S16 = np.int32(16)
    for t in range(_NUM_TABLES):
        order = _ORDERS[t]
        l0 = jnp.zeros_like(tok0[...])
        l1 = jnp.zeros_like(l0)
        l2 = jnp.zeros_like(l0)
        for p in range(order):
            m = mults[t, p]
            m_lo = m & mask16
            m_hi = lax.shift_right_logical(m, _S16)
            tv = toks[p][...]
            p_lo = m_lo * tv
            p_hi = m_hi * tv
            q0 = p_lo & mask16
            mid = lax.shift_right_logical(p_lo, _S16) + (p_hi & mask16)
            q1 = mid & mask16
            q2 = lax.shift_right_logical(p_hi, _S16) + lax.shift_right_logical(mid, _S16)
            l0 = l0 ^ q0
            l1 = l1 ^ q1
            l2 = l2 ^ q2
        b = bias[t]
        l0 = l0 ^ (b & mask16)
        l1 = l1 ^ lax.shift_right_logical(b, _S16)
        p = _PRIMES[t]
        r1 = _mod_prime(l2 * np.int32(65536) + l1, p)
        r2 = _mod_prime(r1 * np.int32(4096), p)
        idx = _mod_prime(r2 * np.int32(16) + l0, p) + np.int32(_OFFSETS[t])
        nrow, ncol = idx.shape
        for r in range(nrow):
            out_ref[pl.ds((t * nrow + r) * ncol, ncol)] = idx[r]


def _compute_indices(tok0, tok1, tok2, mults, bias):
    P, C = tok0.shape
    return pl.pallas_call(
        _hash_kernel,
        out_shape=jax.ShapeDtypeStruct((_NUM_TABLES * P * C,), jnp.int32),
        in_specs=[
            pl.BlockSpec(memory_space=pltpu.VMEM),
            pl.BlockSpec(memory_space=pltpu.VMEM),
            pl.BlockSpec(memory_space=pltpu.VMEM),
            pl.BlockSpec(memory_space=pltpu.SMEM),
            pl.BlockSpec(memory_space=pltpu.SMEM),
        ],
        out_specs=pl.BlockSpec(memory_space=pltpu.VMEM),
        name="ngram_hash",
    )(tok0, tok1, tok2, mults, bias)


_PB = 64
_TG = 8


def _gather_body(bs_total, tableT_hbm, idx_hbm, out_lo_hbm, out_hi_hbm,
                 idx_v, patch_v, outblk_v, sem):
    wid = lax.axis_index("s") * np.int32(_NC) + lax.axis_index("c")
    tc = wid // np.int32(16)
    g = wid % np.int32(16)
    iota16 = lax.iota(jnp.int32, 16)

    def block(_, m):
        b = lax.div(m, np.int32(_TG))
        t8 = lax.rem(m, np.int32(_TG))
        pos0 = lax.add(lax.mul(g, np.int32(512)), lax.mul(b, np.int32(_PB)))
        tglob = lax.add(lax.mul(tc, np.int32(_TG)), t8)
        off = lax.add(lax.mul(tglob, np.int32(bs_total)), pos0)
        pltpu.sync_copy(idx_hbm.at[pl.ds(off, _PB)], idx_v)

        for q in range(_PB // 32):
            for kk in range(2):
                vec = idx_v[pl.ds(q * 32 + kk * 16, 16)]
                for k in range(16):
                    i = vec[k]
                    col0 = pl.multiple_of(
                        lax.mul(lax.div(i, np.int32(128)), np.int32(128)),
                        128)
                    slot = (kk * 16 + k) * 16
                    pltpu.make_async_copy(
                        tableT_hbm.at[:, pl.ds(col0, 128)],
                        patch_v.at[pl.ds(slot, 16)],
                        sem,
                    ).start()
            for k in range(32):
                pltpu.make_async_copy(
                    tableT_hbm.at[:, pl.ds(0, 128)],
                    patch_v.at[pl.ds(k * 16, 16)],
                    sem).wait()
            for kk in range(2):
                vec = idx_v[pl.ds(q * 32 + kk * 16, 16)]
                for k in range(16):
                    i = vec[k]
                    lane = lax.broadcast(lax.rem(i, np.int32(128)), (16,))
                    rows = lax.add(iota16, np.int32((kk * 16 + k) * 16))
                    col = plsc.load_gather(patch_v, [rows, lane])
                    p = q * 32 + kk * 16 + k
                    dst = lax.add(np.int32(p * 128),
                                  lax.mul(t8, np.int32(_EMBED_DIM)))
                    outblk_v[pl.ds(dst, _EMBED_DIM)] = col

        @pl.when(t8 == np.int32(_TG - 1))
        def _():
            flat0 = lax.mul(pos0, np.int32(128))

            @pl.when(tc == np.int32(0))
            def _():
                pltpu.sync_copy(outblk_v,
                                out_lo_hbm.at[pl.ds(flat0, _PB * 128)])

            @pl.when(tc == np.int32(1))
            def _():
                pltpu.sync_copy(outblk_v,
                                out_hi_hbm.at[pl.ds(flat0, _PB * 128)])

        return lax.add(m, np.int32(1))

    lax.fori_loop(0, 8 * _TG, block, np.int32(0))


def _gather_rows(table, idx_flat):
    n = idx_flat.shape[0]
    bs_total = n // _NUM_TABLES
    tableT = table.T
    mesh = plsc.VectorSubcoreMesh(core_axis_name="c", subcore_axis_name="s")
    k = pl.kernel(
        functools.partial(_gather_body, bs_total),
        out_type=[
            jax.ShapeDtypeStruct((bs_total * _TG * _EMBED_DIM,), jnp.float32),
            jax.ShapeDtypeStruct((bs_total * _TG * _EMBED_DIM,), jnp.float32),
        ],
        mesh=mesh,
        name="sc_gather",
        compiler_params=pltpu.CompilerParams(needs_layout_passes=False),
        scratch_types=[
            pltpu.VMEM((_PB,), jnp.int32),
            pltpu.VMEM((32 * 16, 128), jnp.float32),
            pltpu.VMEM((_PB * 128,), jnp.float32),
            pltpu.SemaphoreType.DMA,
        ],
    )
    return k(tableT, idx_flat)


def _matmul_kernel(lo_ref, hi_ref, w_ref, out_ref):
    emb = jnp.concatenate([lo_ref[...], hi_ref[...]], axis=1)
    out_ref[...] = lax.dot_general(
        emb, w_ref[...],
        (((1,), (1,)), ((), ())),
        preferred_element_type=jnp.float32)


def _project(emb_lo, emb_hi, w_out):
    n, half = emb_lo.shape
    blk = 1024
    return pl.pallas_call(
        _matmul_kernel,
        grid=(n // blk,),
        in_specs=[
            pl.BlockSpec((blk, half), lambda i: (i, np.int32(0))),
            pl.BlockSpec((blk, half), lambda i: (i, np.int32(0))),
            pl.BlockSpec((_HIDDEN, _NUM_TABLES * _EMBED_DIM),
                         lambda i: (np.int32(0), np.int32(0))),
        ],
        out_specs=pl.BlockSpec((blk, _HIDDEN), lambda i: (i, np.int32(0))),
        out_shape=jax.ShapeDtypeStruct((n, _HIDDEN), jnp.float32),
        name="out_proj",
    )(emb_lo, emb_hi, w_out)


def kernel(token_ids, hash_mults, hash_bias, table, w_out):
    B, S = token_ids.shape
    P = B * S // 128
    tok0 = token_ids.astype(jnp.int32)
    tok1 = jnp.pad(tok0[:, :S - 1], ((0, 0), (1, 0)))
    tok2 = jnp.pad(tok0[:, :S - 2], ((0, 0), (2, 0)))
    tok0, tok1, tok2 = (t.reshape(P, 128) for t in (tok0, tok1, tok2))
    mults = hash_mults.astype(jnp.int32)
    bias = hash_bias.astype(jnp.int32)

    idx_flat = _compute_indices(tok0, tok1, tok2, mults, bias)
    lo_flat, hi_flat = _gather_rows(table, idx_flat)
    emb_lo = lo_flat.reshape(B * S, _TG * _EMBED_DIM)
    emb_hi = hi_flat.reshape(B * S, _TG * _EMBED_DIM)
    out = _project(emb_lo, emb_hi, w_out)
    return out.reshape(B, S, _HIDDEN)


# speedup vs baseline: 5.6645x; 1.0483x over previous
import functools

import jax
import jax.numpy as jnp
import numpy as np
from jax import lax
from jax.experimental import pallas as pl
from jax.experimental.pallas import tpu as pltpu
from jax.experimental.pallas import tpu_sc as plsc

_PRIMES = (499801, 499819, 499853, 499879, 499883, 499897, 499903, 499927,
           499943, 499957, 499969, 499973, 499979, 500009, 500029, 500041)
_NUM_TABLES = 16
_EMBED_DIM = 16
_HIDDEN = 1024
_ORDERS = tuple([2] * 8 + [3] * 8)
_OFFSETS = tuple(np.concatenate([[0], np.cumsum(_PRIMES)[:-1]]).astype(np.int64))

_NC, _NS = 2, 16
_NW = _NC * _NS


def _mod_prime(x, p):
    q = (x.astype(jnp.float32) * np.float32(1.0 / p)).astype(jnp.int32)
    r = x - q * np.int32(p)
    r = jnp.where(r < 0, r + np.int32(p), r)
    r = jnp.where(r >= np.int32(p), r - np.int32(p), r)
    return r


def _hash_kernel(tok0, tok1, tok2, mults, bias, out_ref):
    toks = (tok0, tok1, tok2)
    mask16 = np.int32(0xFFFF)
    _S16 = np.int32(16)
    for t in range(_NUM_TABLES):
        order = _ORDERS[t]
        l0 = jnp.zeros_like(tok0[...])
        l1 = jnp.zeros_like(l0)
        l2 = jnp.zeros_like(l0)
        for p in range(order):
            m = mults[t, p]
            m_lo = m & mask16
            m_hi = lax.shift_right_logical(m, _S16)
            tv = toks[p][...]
            p_lo = m_lo * tv
            p_hi = m_hi * tv
            q0 = p_lo & mask16
            mid = lax.shift_right_logical(p_lo, _S16) + (p_hi & mask16)
            q1 = mid & mask16
            q2 = lax.shift_right_logical(p_hi, _S16) + lax.shift_right_logical(mid, _S16)
            l0 = l0 ^ q0
            l1 = l1 ^ q1
            l2 = l2 ^ q2
        b = bias[t]
        l0 = l0 ^ (b & mask16)
        l1 = l1 ^ lax.shift_right_logical(b, _S16)
        p = _PRIMES[t]
        r1 = _mod_prime(l2 * np.int32(65536) + l1, p)
        r2 = _mod_prime(r1 * np.int32(4096), p)
        idx = _mod_prime(r2 * np.int32(16) + l0, p) + np.int32(_OFFSETS[t])
        nrow, ncol = idx.shape
        for r in range(nrow):
            out_ref[pl.ds((t * nrow + r) * ncol, ncol)] = idx[r]


def _compute_indices(tok0, tok1, tok2, mults, bias):
    P, C = tok0.shape
    return pl.pallas_call(
        _hash_kernel,
        out_shape=jax.ShapeDtypeStruct((_NUM_TABLES * P * C,), jnp.int32),
        in_specs=[
            pl.BlockSpec(memory_space=pltpu.VMEM),
            pl.BlockSpec(memory_space=pltpu.VMEM),
            pl.BlockSpec(memory_space=pltpu.VMEM),
            pl.BlockSpec(memory_space=pltpu.SMEM),
            pl.BlockSpec(memory_space=pltpu.SMEM),
        ],
        out_specs=pl.BlockSpec(memory_space=pltpu.VMEM),
        name="ngram_hash",
    )(tok0, tok1, tok2, mults, bias)


_PB = 64
_TG = 8


def _gather_body(bs_total, tableT_hbm, idx_hbm, out_lo_hbm, out_hi_hbm,
                 idx_v, patch_v, outblk_v, sem, sem_idx):
    wid = lax.axis_index("s") * np.int32(_NC) + lax.axis_index("c")
    tc = wid // np.int32(16)
    g = wid % np.int32(16)
    iota16 = lax.iota(jnp.int32, 16)

    def idx_off(m):
        b = lax.div(m, np.int32(_TG))
        t8 = lax.rem(m, np.int32(_TG))
        pos0 = lax.add(lax.mul(g, np.int32(512)), lax.mul(b, np.int32(_PB)))
        tglob = lax.add(lax.mul(tc, np.int32(_TG)), t8)
        return lax.add(lax.mul(tglob, np.int32(bs_total)), pos0), pos0, t8

    def idx_fetch(m, half):
        off, _, _ = idx_off(m)
        pltpu.make_async_copy(
            idx_hbm.at[pl.ds(off, _PB)],
            idx_v.at[pl.ds(lax.mul(half, np.int32(_PB)), _PB)],
            sem_idx).start()

    idx_fetch(np.int32(0), np.int32(0))

    def block(_, m):
        _, pos0, t8 = idx_off(m)
        half = lax.rem(m, np.int32(2))
        base = lax.mul(half, np.int32(_PB))
        pltpu.make_async_copy(
            idx_hbm.at[pl.ds(0, _PB)],
            idx_v.at[pl.ds(base, _PB)], sem_idx).wait()

        @pl.when(m < np.int32(8 * _TG - 1))
        def _():
            idx_fetch(lax.add(m, np.int32(1)),
                      lax.rem(lax.add(m, np.int32(1)), np.int32(2)))

        for q in range(_PB // 32):
            for kk in range(2):
                vec = idx_v[pl.ds(lax.add(base, np.int32(q * 32 + kk * 16)),
                                  16)]
                for k in range(16):
                    i = vec[k]
                    col0 = pl.multiple_of(
                        lax.mul(lax.div(i, np.int32(128)), np.int32(128)),
                        128)
                    slot = (kk * 16 + k) * 16
                    pltpu.make_async_copy(
                        tableT_hbm.at[:, pl.ds(col0, 128)],
                        patch_v.at[pl.ds(slot, 16)],
                        sem,
                    ).start()
            for k in range(32):
                pltpu.make_async_copy(
                    tableT_hbm.at[:, pl.ds(0, 128)],
                    patch_v.at[pl.ds(k * 16, 16)],
                    sem).wait()
            for kk in range(2):
                vec = idx_v[pl.ds(lax.add(base, np.int32(q * 32 + kk * 16)),
                                  16)]
                for k in range(16):
                    i = vec[k]
                    lane = lax.broadcast(lax.rem(i, np.int32(128)), (16,))
                    rows = lax.add(iota16, np.int32((kk * 16 + k) * 16))
                    col = plsc.load_gather(patch_v, [rows, lane])
                    p = q * 32 + kk * 16 + k
                    dst = lax.add(np.int32(p * 128),
                                  lax.mul(t8, np.int32(_EMBED_DIM)))
                    outblk_v[pl.ds(dst, _EMBED_DIM)] = col

        @pl.when(t8 == np.int32(_TG - 1))
        def _():
            flat0 = lax.mul(pos0, np.int32(128))

            @pl.when(tc == np.int32(0))
            def _():
                pltpu.sync_copy(outblk_v,
                                out_lo_hbm.at[pl.ds(flat0, _PB * 128)])

            @pl.when(tc == np.int32(1))
            def _():
                pltpu.sync_copy(outblk_v,
                                out_hi_hbm.at[pl.ds(flat0, _PB * 128)])

        return lax.add(m, np.int32(1))

    lax.fori_loop(0, 8 * _TG, block, np.int32(0))


def _gather_rows(table, idx_flat):
    n = idx_flat.shape[0]
    bs_total = n // _NUM_TABLES
    tableT = table.T
    mesh = plsc.VectorSubcoreMesh(core_axis_name="c", subcore_axis_name="s")
    k = pl.kernel(
        functools.partial(_gather_body, bs_total),
        out_type=[
            jax.ShapeDtypeStruct((bs_total * _TG * _EMBED_DIM,), jnp.float32),
            jax.ShapeDtypeStruct((bs_total * _TG * _EMBED_DIM,), jnp.float32),
        ],
        mesh=mesh,
        name="sc_gather",
        compiler_params=pltpu.CompilerParams(needs_layout_passes=False),
        scratch_types=[
            pltpu.VMEM((2 * _PB,), jnp.int32),
            pltpu.VMEM((32 * 16, 128), jnp.float32),
            pltpu.VMEM((_PB * 128,), jnp.float32),
            pltpu.SemaphoreType.DMA,
            pltpu.SemaphoreType.DMA,
        ],
    )
    return k(tableT, idx_flat)


def _matmul_kernel(lo_ref, hi_ref, w_ref, out_ref):
    emb = jnp.concatenate([lo_ref[...], hi_ref[...]], axis=1)
    out_ref[...] = lax.dot_general(
        emb, w_ref[...],
        (((1,), (1,)), ((), ())),
        preferred_element_type=jnp.float32)


def _project(emb_lo, emb_hi, w_out):
    n, half = emb_lo.shape
    blk = 1024
    return pl.pallas_call(
        _matmul_kernel,
        grid=(n // blk,),
        in_specs=[
            pl.BlockSpec((blk, half), lambda i: (i, np.int32(0))),
            pl.BlockSpec((blk, half), lambda i: (i, np.int32(0))),
            pl.BlockSpec((_HIDDEN, _NUM_TABLES * _EMBED_DIM),
                         lambda i: (np.int32(0), np.int32(0))),
        ],
        out_specs=pl.BlockSpec((blk, _HIDDEN), lambda i: (i, np.int32(0))),
        out_shape=jax.ShapeDtypeStruct((n, _HIDDEN), jnp.float32),
        name="out_proj",
    )(emb_lo, emb_hi, w_out)


def kernel(token_ids, hash_mults, hash_bias, table, w_out):
    B, S = token_ids.shape
    P = B * S // 128
    tok0 = token_ids.astype(jnp.int32)
    tok1 = jnp.pad(tok0[:, :S - 1], ((0, 0), (1, 0)))
    tok2 = jnp.pad(tok0[:, :S - 2], ((0, 0), (2, 0)))
    tok0, tok1, tok2 = (t.reshape(P, 128) for t in (tok0, tok1, tok2))
    mults = hash_mults.astype(jnp.int32)
    bias = hash_bias.astype(jnp.int32)

    idx_flat = _compute_indices(tok0, tok1, tok2, mults, bias)
    lo_flat, hi_flat = _gather_rows(table, idx_flat)
    emb_lo = lo_flat.reshape(B * S, _TG * _EMBED_DIM)
    emb_hi = hi_flat.reshape(B * S, _TG * _EMBED_DIM)
    out = _project(emb_lo, emb_hi, w_out)
    return out.reshape(B, S, _HIDDEN)
